# Initial kernel scaffold; baseline (speedup 1.0000x reference)
import jax
import jax.numpy as jnp
from jax.experimental import pallas as pl


def kernel(top_indices, speaker_map, genre_id, genre_emb, distance_emb, speaker_emb):
    raise NotImplementedError("write your pallas kernel here")


# speedup vs baseline: 20.4634x; 20.4634x over previous
import functools

import jax
import jax.numpy as jnp
from jax import lax
from jax.experimental import pallas as pl
from jax.experimental.pallas import tpu as pltpu
from jax.experimental.pallas import tpu_sc as plsc

N_WORDS = 8192
K_ANT = 64
EMB = 32
N_PAIRS = N_WORDS * K_ANT

NUM_CORES = 2
NUM_SUBCORES = 16
NUM_TILES = NUM_CORES * NUM_SUBCORES
WORDS_PER_TILE = N_WORDS // NUM_TILES
PAIRS_PER_TILE = WORDS_PER_TILE * K_ANT
LANES = 16

N_CLASSES = 18


def _sc_classes_body(ti_hbm, spk_hbm, c_hbm, spk_v, ti_v, c_v):
    wid = lax.axis_index("s") * NUM_CORES + lax.axis_index("c")
    base = wid * PAIRS_PER_TILE

    pltpu.sync_copy(spk_hbm, spk_v)
    pltpu.sync_copy(ti_hbm.at[pl.ds(base, PAIRS_PER_TILE)], ti_v)

    def word_body(w, carry):
        i_scalar = wid * WORDS_PER_TILE + w
        i_vec = jnp.full((LANES,), 0, jnp.int32) + i_scalar
        spk_i = plsc.load_gather(spk_v, [i_vec])
        for v in range(K_ANT // LANES):
            off = w * K_ANT + v * LANES
            ant = ti_v[pl.ds(off, LANES)]
            spk_a = plsc.load_gather(spk_v, [ant])
            same = (spk_a == spk_i).astype(jnp.int32)
            dist = jnp.maximum(i_vec - ant, 1)
            lg = (plsc.bitcast(dist.astype(jnp.float32), jnp.int32) >> 23) - 127
            didx = jnp.where(dist < 5, dist - 1, jnp.minimum(lg, 6) + 2)
            c_v[pl.ds(off, LANES)] = same * 9 + didx
        return carry

    lax.fori_loop(0, WORDS_PER_TILE, word_body, 0)
    pltpu.sync_copy(c_v, c_hbm.at[pl.ds(base, PAIRS_PER_TILE)])


def _sc_classes(ti_flat, spk):
    sc = functools.partial(
        pl.kernel,
        out_type=jax.ShapeDtypeStruct((N_PAIRS,), jnp.int32),
        mesh=plsc.VectorSubcoreMesh(core_axis_name="c", subcore_axis_name="s"),
        scratch_types=[
            pltpu.VMEM((N_WORDS,), jnp.int32),
            pltpu.VMEM((PAIRS_PER_TILE,), jnp.int32),
            pltpu.VMEM((PAIRS_PER_TILE,), jnp.int32),
        ],
        compiler_params=pltpu.CompilerParams(needs_layout_passes=False),
    )(_sc_classes_body)
    return sc(ti_flat, spk)


BLK = 2048


def _tc_expand_body(c_ref, tbl_ref, o_ref):
    c = c_ref[...]
    iota = lax.broadcasted_iota(jnp.int32, (BLK, N_CLASSES), 1)
    onehot = (c == iota).astype(jnp.float32)
    o_ref[...] = lax.dot_general(
        onehot, tbl_ref[...], (((1,), (0,)), ((), ())),
        preferred_element_type=jnp.float32,
    )


def _tc_expand(c2d, table):
    return pl.pallas_call(
        _tc_expand_body,
        grid=(N_PAIRS // BLK,),
        in_specs=[
            pl.BlockSpec((BLK, 1), lambda i: (i, 0)),
            pl.BlockSpec((N_CLASSES, 3 * EMB), lambda i: (0, 0)),
        ],
        out_specs=pl.BlockSpec((BLK, 3 * EMB), lambda i: (i, 0)),
        out_shape=jax.ShapeDtypeStruct((N_PAIRS, 3 * EMB), jnp.float32),
    )(c2d, table)


def kernel(top_indices, speaker_map, genre_id, genre_emb, distance_emb, speaker_emb):
    ti_flat = top_indices.reshape(-1).astype(jnp.int32)
    spk = speaker_map.astype(jnp.int32)

    c = _sc_classes(ti_flat, spk)

    genre_row = jnp.take(genre_emb, jnp.asarray(genre_id, jnp.int32)[None], axis=0)
    table = jnp.concatenate(
        [
            jnp.repeat(speaker_emb, 9, axis=0),
            jnp.tile(distance_emb, (2, 1)),
            jnp.broadcast_to(genre_row, (N_CLASSES, EMB)),
        ],
        axis=1,
    )

    out2d = _tc_expand(c.reshape(N_PAIRS, 1), table)
    return out2d.reshape(N_WORDS, K_ANT, 3 * EMB)


# speedup vs baseline: 20.8866x; 1.0207x over previous
import functools

import jax
import jax.numpy as jnp
from jax import lax
from jax.experimental import pallas as pl
from jax.experimental.pallas import tpu as pltpu
from jax.experimental.pallas import tpu_sc as plsc

N_WORDS = 8192
K_ANT = 64
EMB = 32
N_PAIRS = N_WORDS * K_ANT

NUM_CORES = 2
NUM_SUBCORES = 16
NUM_TILES = NUM_CORES * NUM_SUBCORES
WORDS_PER_TILE = N_WORDS // NUM_TILES
PAIRS_PER_TILE = WORDS_PER_TILE * K_ANT
LANES = 16

N_CLASSES = 18


def _sc_classes_body(ti_hbm, spk_hbm, c_hbm, spk_v, ti_v, c_v):
    wid = lax.axis_index("s") * NUM_CORES + lax.axis_index("c")
    base = wid * PAIRS_PER_TILE

    pltpu.sync_copy(spk_hbm, spk_v)
    pltpu.sync_copy(ti_hbm.at[pl.ds(base, PAIRS_PER_TILE)], ti_v)

    def word_body(w, carry):
        i_scalar = wid * WORDS_PER_TILE + w
        i_vec = jnp.full((LANES,), 0, jnp.int32) + i_scalar
        spk_i = plsc.load_gather(spk_v, [i_vec])
        for v in range(K_ANT // LANES):
            off = w * K_ANT + v * LANES
            ant = ti_v[pl.ds(off, LANES)]
            spk_a = plsc.load_gather(spk_v, [ant])
            same = (spk_a == spk_i).astype(jnp.int32)
            dist = jnp.maximum(i_vec - ant, 1)
            lg = (plsc.bitcast(dist.astype(jnp.float32), jnp.int32) >> 23) - 127
            didx = jnp.where(dist < 5, dist - 1, jnp.minimum(lg, 6) + 2)
            c_v[pl.ds(off, LANES)] = same * 9 + didx
        return carry

    lax.fori_loop(0, WORDS_PER_TILE, word_body, 0)
    pltpu.sync_copy(c_v, c_hbm.at[pl.ds(base, PAIRS_PER_TILE)])


def _sc_classes(ti_flat, spk):
    sc = functools.partial(
        pl.kernel,
        out_type=jax.ShapeDtypeStruct((N_PAIRS,), jnp.int32),
        mesh=plsc.VectorSubcoreMesh(core_axis_name="c", subcore_axis_name="s"),
        scratch_types=[
            pltpu.VMEM((N_WORDS,), jnp.int32),
            pltpu.VMEM((PAIRS_PER_TILE,), jnp.int32),
            pltpu.VMEM((PAIRS_PER_TILE,), jnp.int32),
        ],
        compiler_params=pltpu.CompilerParams(needs_layout_passes=False),
    )(_sc_classes_body)
    return sc(ti_flat, spk)


BLK = 2048


def _tc_expand_body(c_ref, tbl_ref, o_ref):
    c = c_ref[...]
    iota = lax.broadcasted_iota(jnp.int16, (BLK, N_CLASSES), 1)
    onehot = jnp.where(c.astype(jnp.int16) == iota,
                       jnp.bfloat16(1), jnp.bfloat16(0))
    o_ref[...] = lax.dot_general(
        onehot, tbl_ref[...], (((1,), (0,)), ((), ())),
        preferred_element_type=jnp.float32,
    )


def _tc_expand(c2d, table):
    return pl.pallas_call(
        _tc_expand_body,
        grid=(N_PAIRS // BLK,),
        in_specs=[
            pl.BlockSpec((BLK, 1), lambda i: (i, 0)),
            pl.BlockSpec((N_CLASSES, 3 * EMB), lambda i: (0, 0)),
        ],
        compiler_params=pltpu.CompilerParams(
            dimension_semantics=("arbitrary",),
        ),
        out_specs=pl.BlockSpec((BLK, 3 * EMB), lambda i: (i, 0)),
        out_shape=jax.ShapeDtypeStruct((N_PAIRS, 3 * EMB), jnp.float32),
    )(c2d, table)


def kernel(top_indices, speaker_map, genre_id, genre_emb, distance_emb, speaker_emb):
    ti_flat = top_indices.reshape(-1).astype(jnp.int32)
    spk = speaker_map.astype(jnp.int32)

    c = _sc_classes(ti_flat, spk)

    genre_row = jnp.take(genre_emb, jnp.asarray(genre_id, jnp.int32)[None], axis=0)
    table = jnp.concatenate(
        [
            jnp.repeat(speaker_emb, 9, axis=0),
            jnp.tile(distance_emb, (2, 1)),
            jnp.broadcast_to(genre_row, (N_CLASSES, EMB)),
        ],
        axis=1,
    )

    out2d = _tc_expand(c.reshape(N_PAIRS, 1), table.astype(jnp.bfloat16))
    return out2d.reshape(N_WORDS, K_ANT, 3 * EMB)


# speedup vs baseline: 43.8433x; 2.0991x over previous
import functools

import jax
import jax.numpy as jnp
from jax import lax
from jax.experimental import pallas as pl
from jax.experimental.pallas import tpu as pltpu
from jax.experimental.pallas import tpu_sc as plsc

N_WORDS = 8192
K_ANT = 64
EMB = 32
N_PAIRS = N_WORDS * K_ANT

NUM_CORES = 2
NUM_SUBCORES = 16
NUM_TILES = NUM_CORES * NUM_SUBCORES
WORDS_PER_TILE = N_WORDS // NUM_TILES
PAIRS_PER_TILE = WORDS_PER_TILE * K_ANT
LANES = 16

N_CLASSES = 18


def _sc_classes_body(ti_hbm, spk_hbm, c_hbm, spk_v, ti_v, c_v):
    wid = lax.axis_index("s") * NUM_CORES + lax.axis_index("c")
    base = wid * PAIRS_PER_TILE

    pltpu.sync_copy(spk_hbm, spk_v)
    pltpu.sync_copy(ti_hbm.at[pl.ds(base, PAIRS_PER_TILE)], ti_v)

    def word_body(w, carry):
        i_scalar = wid * WORDS_PER_TILE + w
        i_vec = jnp.full((LANES,), 0, jnp.int32) + i_scalar
        spk_i = plsc.load_gather(spk_v, [i_vec])
        for v in range(K_ANT // LANES):
            off = w * K_ANT + v * LANES
            ant = ti_v[pl.ds(off, LANES)]
            spk_a = plsc.load_gather(spk_v, [ant])
            same = (spk_a == spk_i).astype(jnp.int32)
            dist = jnp.maximum(i_vec - ant, 1)
            lg = (plsc.bitcast(dist.astype(jnp.float32), jnp.int32) >> 23) - 127
            didx = jnp.where(dist < 5, dist - 1, jnp.minimum(lg, 6) + 2)
            c_v[pl.ds(off, LANES)] = same * 9 + didx
        return carry

    lax.fori_loop(0, WORDS_PER_TILE, word_body, 0)
    pltpu.sync_copy(c_v, c_hbm.at[pl.ds(base, PAIRS_PER_TILE)])


def _sc_classes(ti_flat, spk):
    sc = functools.partial(
        pl.kernel,
        out_type=jax.ShapeDtypeStruct((N_PAIRS,), jnp.int32),
        mesh=plsc.VectorSubcoreMesh(core_axis_name="c", subcore_axis_name="s"),
        scratch_types=[
            pltpu.VMEM((N_WORDS,), jnp.int32),
            pltpu.VMEM((PAIRS_PER_TILE,), jnp.int32),
            pltpu.VMEM((PAIRS_PER_TILE,), jnp.int32),
        ],
        compiler_params=pltpu.CompilerParams(needs_layout_passes=False),
    )(_sc_classes_body)
    return sc(ti_flat, spk)


BLK_ROWS = 16
BLK = BLK_ROWS * 128
C_ROWS = N_PAIRS // 128


def _tc_expand_body(c_ref, tbl_ref, o_ref):
    cb = c_ref[...].astype(jnp.int16)
    iota = lax.broadcasted_iota(jnp.int16, (N_CLASSES, 128), 0)
    t = tbl_ref[...]
    for r in range(BLK_ROWS):
        row = lax.broadcast_in_dim(cb[r], (N_CLASSES, 128), (1,))
        m = jnp.where(row == iota, jnp.bfloat16(1), jnp.bfloat16(0))
        o_ref[r * 128:(r + 1) * 128, :] = lax.dot_general(
            m, t, (((0,), (0,)), ((), ())),
            preferred_element_type=jnp.float32)


def _tc_expand(c2, table):
    return pl.pallas_call(
        _tc_expand_body,
        grid=(C_ROWS // BLK_ROWS,),
        in_specs=[
            pl.BlockSpec((BLK_ROWS, 128), lambda i: (i, 0)),
            pl.BlockSpec((N_CLASSES, 3 * EMB), lambda i: (0, 0)),
        ],
        compiler_params=pltpu.CompilerParams(
            dimension_semantics=("arbitrary",),
        ),
        out_specs=pl.BlockSpec((BLK, 3 * EMB), lambda i: (i, 0)),
        out_shape=jax.ShapeDtypeStruct((N_PAIRS, 3 * EMB), jnp.float32),
    )(c2, table)


def kernel(top_indices, speaker_map, genre_id, genre_emb, distance_emb, speaker_emb):
    ti_flat = top_indices.reshape(-1).astype(jnp.int32)
    spk = speaker_map.astype(jnp.int32)

    c = _sc_classes(ti_flat, spk)

    genre_row = jnp.take(genre_emb, jnp.asarray(genre_id, jnp.int32)[None], axis=0)
    table = jnp.concatenate(
        [
            jnp.repeat(speaker_emb, 9, axis=0),
            jnp.tile(distance_emb, (2, 1)),
            jnp.broadcast_to(genre_row, (N_CLASSES, EMB)),
        ],
        axis=1,
    )

    out2d = _tc_expand(c.reshape(C_ROWS, 128), table.astype(jnp.bfloat16))
    return out2d.reshape(N_WORDS, K_ANT, 3 * EMB)
